# Initial kernel scaffold; baseline (speedup 1.0000x reference)
import jax
import jax.numpy as jnp
from jax.experimental import pallas as pl


def kernel(x, flat_pe, segment_ids):
    raise NotImplementedError("write your pallas kernel here")


# speedup vs baseline: 4.5424x; 4.5424x over previous
import jax
import jax.numpy as jnp
from jax.experimental import pallas as pl
from jax.experimental.pallas import tpu as pltpu

_B = 16
_S = 4096
_D = 1024
_PE = 64
_N = 32768
_BS = 512


def _pe_add_kernel(seg_ref, pe_ref, x_ref, o_ref):
    b = pl.program_id(0)
    s = pl.program_id(1)
    p0 = s * _BS
    seg = seg_ref[...]
    start_b = jnp.sum((seg < b).astype(jnp.int32))
    len_b = jnp.sum((seg == b).astype(jnp.int32))
    j0 = start_b + p0
    pe_blk = pe_ref[pl.ds(j0, _BS), :]
    p = p0 + jax.lax.broadcasted_iota(jnp.int32, (_BS, 1), 0)
    valid = (p >= 1) & (p <= len_b)
    pe_blk = jnp.where(valid, pe_blk, 0.0)
    xb = x_ref[0]
    o_ref[0, :, :_PE] = xb[:, :_PE] + pe_blk
    o_ref[0, :, _PE:] = xb[:, _PE:] + pe_blk[:, _PE - 1:_PE]


def kernel(x, flat_pe, segment_ids):
    pe_ext = jnp.concatenate(
        [jnp.zeros((1, _PE), x.dtype), flat_pe, jnp.zeros((_S - 1, _PE), x.dtype)]
    )
    seg2d = segment_ids.reshape(8, _N // 8)
    grid = (_B, _S // _BS)
    return pl.pallas_call(
        _pe_add_kernel,
        grid=grid,
        in_specs=[
            pl.BlockSpec((8, _N // 8), lambda b, s: (0, 0)),
            pl.BlockSpec((_N + _S, _PE), lambda b, s: (0, 0)),
            pl.BlockSpec((1, _BS, _D), lambda b, s: (b, s, 0)),
        ],
        out_specs=pl.BlockSpec((1, _BS, _D), lambda b, s: (b, s, 0)),
        out_shape=jax.ShapeDtypeStruct(x.shape, x.dtype),
        compiler_params=pltpu.CompilerParams(
            dimension_semantics=("parallel", "parallel"),
        ),
    )(seg2d, pe_ext, x)


# speedup vs baseline: 4.9994x; 1.1006x over previous
import jax
import jax.numpy as jnp
from jax.experimental import pallas as pl
from jax.experimental.pallas import tpu as pltpu

_B = 16
_S = 4096
_D = 1024
_PE = 64
_N = 32768
_BS = 1024


def _pe_add_kernel(seg_ref, pe_ref, x_ref, o_ref):
    b = pl.program_id(0)
    s = pl.program_id(1)
    p0 = s * _BS
    seg = seg_ref[...]
    start_b = jnp.sum((seg < b).astype(jnp.int32))
    len_b = jnp.sum((seg == b).astype(jnp.int32))
    j0 = start_b + p0
    pe_blk = pe_ref[pl.ds(j0, _BS), :]
    p = p0 + jax.lax.broadcasted_iota(jnp.int32, (_BS, 1), 0)
    valid = (p >= 1) & (p <= len_b)
    pe_blk = jnp.where(valid, pe_blk, 0.0)
    xb = x_ref[0]
    o_ref[0, :, :_PE] = xb[:, :_PE] + pe_blk
    o_ref[0, :, _PE:] = xb[:, _PE:] + pe_blk[:, _PE - 1:_PE]


def kernel(x, flat_pe, segment_ids):
    pe_ext = jnp.concatenate(
        [jnp.zeros((1, _PE), x.dtype), flat_pe, jnp.zeros((_S - 1, _PE), x.dtype)]
    )
    seg2d = segment_ids.reshape(8, _N // 8)
    grid = (_B, _S // _BS)
    return pl.pallas_call(
        _pe_add_kernel,
        grid=grid,
        in_specs=[
            pl.BlockSpec((8, _N // 8), lambda b, s: (0, 0)),
            pl.BlockSpec((_N + _S, _PE), lambda b, s: (0, 0)),
            pl.BlockSpec((1, _BS, _D), lambda b, s: (b, s, 0)),
        ],
        out_specs=pl.BlockSpec((1, _BS, _D), lambda b, s: (b, s, 0)),
        out_shape=jax.ShapeDtypeStruct(x.shape, x.dtype),
        compiler_params=pltpu.CompilerParams(
            dimension_semantics=("parallel", "parallel"),
        ),
    )(seg2d, pe_ext, x)


# speedup vs baseline: 5.0475x; 1.0096x over previous
import jax
import jax.numpy as jnp
from jax.experimental import pallas as pl
from jax.experimental.pallas import tpu as pltpu

_B = 16
_S = 4096
_D = 1024
_PE = 64
_N = 32768
_BS = 2048


def _pe_add_kernel(seg_ref, pe_ref, x_ref, o_ref):
    b = pl.program_id(0)
    s = pl.program_id(1)
    p0 = s * _BS
    seg = seg_ref[...]
    start_b = jnp.sum((seg < b).astype(jnp.int32))
    len_b = jnp.sum((seg == b).astype(jnp.int32))
    j0 = start_b + p0
    pe_blk = pe_ref[pl.ds(j0, _BS), :]
    p = p0 + jax.lax.broadcasted_iota(jnp.int32, (_BS, 1), 0)
    valid = (p >= 1) & (p <= len_b)
    pe_blk = jnp.where(valid, pe_blk, 0.0)
    xb = x_ref[0]
    o_ref[0, :, :_PE] = xb[:, :_PE] + pe_blk
    o_ref[0, :, _PE:] = xb[:, _PE:] + pe_blk[:, _PE - 1:_PE]


def kernel(x, flat_pe, segment_ids):
    pe_ext = jnp.concatenate(
        [jnp.zeros((1, _PE), x.dtype), flat_pe, jnp.zeros((_S - 1, _PE), x.dtype)]
    )
    seg2d = segment_ids.reshape(8, _N // 8)
    grid = (_B, _S // _BS)
    return pl.pallas_call(
        _pe_add_kernel,
        grid=grid,
        in_specs=[
            pl.BlockSpec((8, _N // 8), lambda b, s: (0, 0)),
            pl.BlockSpec((_N + _S, _PE), lambda b, s: (0, 0)),
            pl.BlockSpec((1, _BS, _D), lambda b, s: (b, s, 0)),
        ],
        out_specs=pl.BlockSpec((1, _BS, _D), lambda b, s: (b, s, 0)),
        out_shape=jax.ShapeDtypeStruct(x.shape, x.dtype),
        compiler_params=pltpu.CompilerParams(
            dimension_semantics=("parallel", "parallel"),
        ),
    )(seg2d, pe_ext, x)
